# Initial kernel scaffold; baseline (speedup 1.0000x reference)
import functools

import jax
import jax.numpy as jnp
from jax import lax
from jax.experimental import pallas as pl
from jax.experimental.pallas import tpu as pltpu
from jax.experimental.pallas import tpu_sc as plsc

SIGMA = 0.05

B = 64
OBS = 2048
HID = 1024
ACT = 8192
NS = 100

ACT_BLK = 1024
N_ACT_BLK = ACT // ACT_BLK

NWORKERS = 32
B_PER_W = B // NWORKERS
CHUNKS = ACT // 16


def _policy_body(obs_ref, W1_ref, b1_ref, W2_ref, b2_ref, Wv1a_ref, bv1_ref,
                 logits_ref, c0_ref, h_scr, s_scr):
    k = pl.program_id(0)

    @pl.when(k == 0)
    def _():
        h_scr[:] = jnp.tanh(
            jnp.dot(obs_ref[:], W1_ref[:], preferred_element_type=jnp.float32)
            + b1_ref[:])
        c0_ref[:] = (
            jnp.dot(obs_ref[:], Wv1a_ref[:], preferred_element_type=jnp.float32)
            + bv1_ref[:])

    s_scr[:, pl.ds(k * ACT_BLK, ACT_BLK)] = (
        jnp.dot(h_scr[:], W2_ref[:], preferred_element_type=jnp.float32)
        + b2_ref[:])

    @pl.when(k == N_ACT_BLK - 1)
    def _():
        s = s_scr[:]
        m = jnp.max(s, axis=1, keepdims=True)
        e = jnp.exp(s - m)
        logits_ref[:] = e / jnp.sum(e, axis=1, keepdims=True)


def _policy(obs, W1, b1, W2, b2, Wv1a, bv1):
    return pl.pallas_call(
        _policy_body,
        grid=(N_ACT_BLK,),
        in_specs=[
            pl.BlockSpec((B, OBS), lambda k: (0, 0)),
            pl.BlockSpec((OBS, HID), lambda k: (0, 0)),
            pl.BlockSpec((1, HID), lambda k: (0, 0)),
            pl.BlockSpec((HID, ACT_BLK), lambda k: (0, k)),
            pl.BlockSpec((1, ACT_BLK), lambda k: (0, k)),
            pl.BlockSpec((OBS, HID), lambda k: (0, 0)),
            pl.BlockSpec((1, HID), lambda k: (0, 0)),
        ],
        out_specs=[
            pl.BlockSpec((B, ACT), lambda k: (0, 0)),
            pl.BlockSpec((B, HID), lambda k: (0, 0)),
        ],
        out_shape=[
            jax.ShapeDtypeStruct((B, ACT), jnp.float32),
            jax.ShapeDtypeStruct((B, HID), jnp.float32),
        ],
        scratch_shapes=[
            pltpu.VMEM((B, HID), jnp.float32),
            pltpu.VMEM((B, ACT), jnp.float32),
        ],
    )(obs, W1, b1, W2, b2, Wv1a, bv1)


def _sc_body(logits_hbm, noise_hbm, out_hbm, logits_v, noise_v, out_v, sem):
    c = lax.axis_index("c")
    s = lax.axis_index("s")
    wid = s * 2 + c
    base_iota = lax.iota(jnp.int32, (16,))

    def batch_body(bi, total_b):
        b = wid * B_PER_W + bi
        pltpu.sync_copy(logits_hbm.at[b], logits_v)

        def row_body(n, tot):
            pltpu.sync_copy(noise_hbm.at[b, n], noise_v)

            def chunk(j, carry):
                bv, bidx = carry
                off = j * 16
                val = logits_v[pl.ds(off, 16)] + SIGMA * noise_v[pl.ds(off, 16)]
                pred = val > bv
                return (jnp.where(pred, val, bv),
                        jnp.where(pred, base_iota + off, bidx))

            bv0 = jnp.full((16,), -3.0e38, jnp.float32)
            bi0 = jnp.zeros((16,), jnp.int32)
            bv, bidx = lax.fori_loop(0, CHUNKS, chunk, (bv0, bi0))
            m = jnp.max(bv)
            cand = jnp.where(bv == m, bidx, jnp.int32(1 << 30))
            return tot + jnp.min(cand)

        return lax.fori_loop(0, NS, row_body, total_b)

    total = lax.fori_loop(0, B_PER_W, batch_body, jnp.int32(0))
    out_v[:] = jnp.where(base_iota == 0, total, 0)
    pltpu.sync_copy(out_v, out_hbm.at[wid])


@functools.partial(
    pl.kernel,
    out_type=jax.ShapeDtypeStruct((NWORKERS, 16), jnp.int32),
    mesh=plsc.VectorSubcoreMesh(core_axis_name="c", subcore_axis_name="s"),
    scratch_types=[
        pltpu.VMEM((ACT,), jnp.float32),
        pltpu.VMEM((ACT,), jnp.float32),
        pltpu.VMEM((16,), jnp.int32),
        pltpu.SemaphoreType.DMA,
    ],
)
def _sc_argmax_sum(logits_hbm, noise_hbm, out_hbm, logits_v, noise_v, out_v, sem):
    _sc_body(logits_hbm, noise_hbm, out_hbm, logits_v, noise_v, out_v, sem)


def _value_body(c0_ref, wlast_ref, ws_ref, Wv2_ref, bv2_ref, q_ref):
    total = jnp.sum(ws_ref[:].astype(jnp.float32)) / jnp.float32(NS)
    hv = jnp.tanh(c0_ref[:] + total * wlast_ref[:])
    q_ref[:] = (jnp.dot(hv, Wv2_ref[:], preferred_element_type=jnp.float32)
                + bv2_ref[:])


def _value(c0, wlast, ws, Wv2, bv2):
    return pl.pallas_call(
        _value_body,
        out_shape=jax.ShapeDtypeStruct((B, 1), jnp.float32),
    )(c0, wlast, ws, Wv2, bv2)


def kernel(obs, W1, b1, W2, b2, Wv1, bv1, Wv2, bv2, noise):
    logits, c0 = _policy(obs, W1, b1.reshape(1, HID), W2, b2.reshape(1, ACT),
                         Wv1[:OBS], bv1.reshape(1, HID))
    ws = _sc_argmax_sum(logits, noise)
    return _value(c0, Wv1[OBS:OBS + 1], ws, Wv2, bv2.reshape(1, 1))


# speedup vs baseline: 31.6627x; 31.6627x over previous
import functools

import jax
import jax.numpy as jnp
from jax import lax
from jax.experimental import pallas as pl
from jax.experimental.pallas import tpu as pltpu
from jax.experimental.pallas import tpu_sc as plsc

SIGMA = 0.05

B = 64
OBS = 2048
HID = 1024
ACT = 8192
NS = 100

ACT_BLK = 1024
N_ACT_BLK = ACT // ACT_BLK

NWORKERS = 32
B_PER_W = B // NWORKERS
CHUNKS = ACT // 16


def _policy_body(obs_ref, W1_ref, b1_ref, W2_ref, b2_ref, Wv1a_ref, bv1_ref,
                 logits_ref, c0_ref, h_scr, s_scr):
    k = pl.program_id(0)

    @pl.when(k == 0)
    def _():
        h_scr[:] = jnp.tanh(
            jnp.dot(obs_ref[:], W1_ref[:], preferred_element_type=jnp.float32)
            + b1_ref[:])
        c0_ref[:] = (
            jnp.dot(obs_ref[:], Wv1a_ref[:], preferred_element_type=jnp.float32)
            + bv1_ref[:])

    s_scr[:, pl.ds(k * ACT_BLK, ACT_BLK)] = (
        jnp.dot(h_scr[:], W2_ref[:], preferred_element_type=jnp.float32)
        + b2_ref[:])

    @pl.when(k == N_ACT_BLK - 1)
    def _():
        s = s_scr[:]
        m = jnp.max(s, axis=1, keepdims=True)
        e = jnp.exp(s - m)
        logits_ref[:] = e / jnp.sum(e, axis=1, keepdims=True)


def _policy(obs, W1, b1, W2, b2, Wv1a, bv1):
    return pl.pallas_call(
        _policy_body,
        grid=(N_ACT_BLK,),
        in_specs=[
            pl.BlockSpec((B, OBS), lambda k: (0, 0)),
            pl.BlockSpec((OBS, HID), lambda k: (0, 0)),
            pl.BlockSpec((1, HID), lambda k: (0, 0)),
            pl.BlockSpec((HID, ACT_BLK), lambda k: (0, k)),
            pl.BlockSpec((1, ACT_BLK), lambda k: (0, k)),
            pl.BlockSpec((OBS, HID), lambda k: (0, 0)),
            pl.BlockSpec((1, HID), lambda k: (0, 0)),
        ],
        out_specs=[
            pl.BlockSpec((B, ACT), lambda k: (0, 0)),
            pl.BlockSpec((B, HID), lambda k: (0, 0)),
        ],
        out_shape=[
            jax.ShapeDtypeStruct((B, ACT), jnp.float32),
            jax.ShapeDtypeStruct((B, HID), jnp.float32),
        ],
        scratch_shapes=[
            pltpu.VMEM((B, HID), jnp.float32),
            pltpu.VMEM((B, ACT), jnp.float32),
        ],
    )(obs, W1, b1, W2, b2, Wv1a, bv1)


_GDN = lax.GatherDimensionNumbers(
    offset_dims=(), collapsed_slice_dims=(0,), start_index_map=(0,))


def _xlane(v, perm):
    return lax.gather(v, perm[:, None], _GDN, slice_sizes=(1,),
                      mode=lax.GatherScatterMode.PROMISE_IN_BOUNDS)


def _sc_body(logits_hbm, noise_hbm, out_hbm, logits_v, noise_v, out_v, sem):
    c = lax.axis_index("c")
    s = lax.axis_index("s")
    wid = s * 2 + c
    base_iota = lax.iota(jnp.int32, 16)

    def batch_body(bi, total_b):
        b = wid * B_PER_W + bi
        pltpu.sync_copy(logits_hbm.at[b], logits_v)

        def row_body(n, tot_vec):
            pltpu.sync_copy(noise_hbm.at[b, n], noise_v)

            def chunk(j, carry):
                bv, bidx = carry
                off = j * 16
                val = logits_v[pl.ds(off, 16)] + SIGMA * noise_v[pl.ds(off, 16)]
                pred = val > bv
                return (jnp.where(pred, val, bv),
                        jnp.where(pred, base_iota + off, bidx))

            bv0 = jnp.full((16,), -3.0e38, jnp.float32)
            bi0 = jnp.zeros((16,), jnp.int32)
            v, i = lax.fori_loop(0, CHUNKS, chunk, (bv0, bi0))
            for shift in (1, 2, 4, 8):
                perm = base_iota ^ shift
                v2 = _xlane(v, perm)
                i2 = _xlane(i, perm)
                better = (v2 > v) | ((v2 == v) & (i2 < i))
                v = jnp.where(better, v2, v)
                i = jnp.where(better, i2, i)
            return tot_vec + jnp.where(base_iota == 0, i, 0)

        return lax.fori_loop(0, NS, row_body, total_b)

    total = lax.fori_loop(0, B_PER_W, batch_body, jnp.zeros((16,), jnp.int32))
    out_v[:] = total
    pltpu.sync_copy(out_v, out_hbm.at[wid])


@functools.partial(
    pl.kernel,
    out_type=jax.ShapeDtypeStruct((NWORKERS, 16), jnp.int32),
    mesh=plsc.VectorSubcoreMesh(core_axis_name="c", subcore_axis_name="s"),
    scratch_types=[
        pltpu.VMEM((ACT,), jnp.float32),
        pltpu.VMEM((ACT,), jnp.float32),
        pltpu.VMEM((16,), jnp.int32),
        pltpu.SemaphoreType.DMA,
    ],
)
def _sc_argmax_sum(logits_hbm, noise_hbm, out_hbm, logits_v, noise_v, out_v, sem):
    _sc_body(logits_hbm, noise_hbm, out_hbm, logits_v, noise_v, out_v, sem)


def _value_body(c0_ref, wlast_ref, ws_ref, Wv2_ref, bv2_ref, q_ref):
    total = jnp.sum(ws_ref[:].astype(jnp.float32)) / jnp.float32(NS)
    hv = jnp.tanh(c0_ref[:] + total * wlast_ref[:])
    q_ref[:] = (jnp.dot(hv, Wv2_ref[:], preferred_element_type=jnp.float32)
                + bv2_ref[:])


def _value(c0, wlast, ws, Wv2, bv2):
    return pl.pallas_call(
        _value_body,
        out_shape=jax.ShapeDtypeStruct((B, 1), jnp.float32),
    )(c0, wlast, ws, Wv2, bv2)


def kernel(obs, W1, b1, W2, b2, Wv1, bv1, Wv2, bv2, noise):
    logits, c0 = _policy(obs, W1, b1.reshape(1, HID), W2, b2.reshape(1, ACT),
                         Wv1[:OBS], bv1.reshape(1, HID))
    ws = _sc_argmax_sum(logits, noise)
    return _value(c0, Wv1[OBS:OBS + 1], ws, Wv2, bv2.reshape(1, 1))


# speedup vs baseline: 61.7152x; 1.9491x over previous
import functools

import jax
import jax.numpy as jnp
from jax import lax
from jax.experimental import pallas as pl
from jax.experimental.pallas import tpu as pltpu
from jax.experimental.pallas import tpu_sc as plsc

SIGMA = 0.05

B = 64
OBS = 2048
HID = 1024
ACT = 8192
NS = 100

ACT_BLK = 1024
N_ACT_BLK = ACT // ACT_BLK

NWORKERS = 32
B_PER_W = B // NWORKERS
CHUNKS = ACT // 16


def _policy_body(obs_ref, W1_ref, b1_ref, W2_ref, b2_ref, Wv1a_ref, bv1_ref,
                 logits_ref, c0_ref, h_scr, s_scr):
    k = pl.program_id(0)

    @pl.when(k == 0)
    def _():
        h_scr[:] = jnp.tanh(
            jnp.dot(obs_ref[:], W1_ref[:], preferred_element_type=jnp.float32)
            + b1_ref[:])
        c0_ref[:] = (
            jnp.dot(obs_ref[:], Wv1a_ref[:], preferred_element_type=jnp.float32)
            + bv1_ref[:])

    s_scr[:, pl.ds(k * ACT_BLK, ACT_BLK)] = (
        jnp.dot(h_scr[:], W2_ref[:], preferred_element_type=jnp.float32)
        + b2_ref[:])

    @pl.when(k == N_ACT_BLK - 1)
    def _():
        s = s_scr[:]
        m = jnp.max(s, axis=1, keepdims=True)
        e = jnp.exp(s - m)
        logits_ref[:] = e / jnp.sum(e, axis=1, keepdims=True)


def _policy(obs, W1, b1, W2, b2, Wv1a, bv1):
    return pl.pallas_call(
        _policy_body,
        grid=(N_ACT_BLK,),
        in_specs=[
            pl.BlockSpec((B, OBS), lambda k: (0, 0)),
            pl.BlockSpec((OBS, HID), lambda k: (0, 0)),
            pl.BlockSpec((1, HID), lambda k: (0, 0)),
            pl.BlockSpec((HID, ACT_BLK), lambda k: (0, k)),
            pl.BlockSpec((1, ACT_BLK), lambda k: (0, k)),
            pl.BlockSpec((OBS, HID), lambda k: (0, 0)),
            pl.BlockSpec((1, HID), lambda k: (0, 0)),
        ],
        out_specs=[
            pl.BlockSpec((B, ACT), lambda k: (0, 0)),
            pl.BlockSpec((B, HID), lambda k: (0, 0)),
        ],
        out_shape=[
            jax.ShapeDtypeStruct((B, ACT), jnp.float32),
            jax.ShapeDtypeStruct((B, HID), jnp.float32),
        ],
        scratch_shapes=[
            pltpu.VMEM((B, HID), jnp.float32),
            pltpu.VMEM((B, ACT), jnp.float32),
        ],
    )(obs, W1, b1, W2, b2, Wv1a, bv1)


_GDN = lax.GatherDimensionNumbers(
    offset_dims=(), collapsed_slice_dims=(0,), start_index_map=(0,))


def _xlane(v, perm):
    return lax.gather(v, perm[:, None], _GDN, slice_sizes=(1,),
                      mode=lax.GatherScatterMode.PROMISE_IN_BOUNDS)


ROWS = 4
GROUPS = NS // ROWS
PAIRS = GROUPS // 2


def _sc_body(logits_hbm, noise_hbm, out_hbm, logits_v, noise_v, out_v,
             sem0, sem1):
    c = lax.axis_index("c")
    s = lax.axis_index("s")
    wid = s * 2 + c
    base_iota = lax.iota(jnp.int32, 16)

    def scan_group(slot, tot):
        def chunk(j, carry):
            off = j * 16
            lv = logits_v[pl.ds(off, 16)]
            cur = base_iota + off
            out = []
            for r in range(ROWS):
                bv, bidx = carry[2 * r], carry[2 * r + 1]
                val = lv + SIGMA * noise_v[slot * ROWS + r, pl.ds(off, 16)]
                pred = val > bv
                out.append(jnp.where(pred, val, bv))
                out.append(jnp.where(pred, cur, bidx))
            return tuple(out)

        init = []
        for _ in range(ROWS):
            init.append(jnp.full((16,), -3.0e38, jnp.float32))
            init.append(jnp.zeros((16,), jnp.int32))
        fin = lax.fori_loop(0, CHUNKS, chunk, tuple(init))
        for r in range(ROWS):
            v, i = fin[2 * r], fin[2 * r + 1]
            for shift in (1, 2, 4, 8):
                perm = base_iota ^ shift
                v2 = _xlane(v, perm)
                i2 = _xlane(i, perm)
                better = (v2 > v) | ((v2 == v) & (i2 < i))
                v = jnp.where(better, v2, v)
                i = jnp.where(better, i2, i)
            tot = tot + jnp.where(base_iota == 0, i, 0)
        return tot

    total = jnp.zeros((16,), jnp.int32)
    for bi in range(B_PER_W):
        b = wid * B_PER_W + bi
        pltpu.sync_copy(logits_hbm.at[b], logits_v)

        def start_group(g, slot, sem):
            for r in range(ROWS):
                pltpu.make_async_copy(
                    noise_hbm.at[b, g * ROWS + r], noise_v.at[slot * ROWS + r],
                    sem).start()

        def wait_group(g, slot, sem):
            for r in range(ROWS):
                pltpu.make_async_copy(
                    noise_hbm.at[b, g * ROWS + r], noise_v.at[slot * ROWS + r],
                    sem).wait()

        start_group(0, 0, sem0)
        start_group(1, 1, sem1)

        def pair_iter(gg, tot):
            g0 = 2 * gg
            wait_group(g0, 0, sem0)
            tot = scan_group(0, tot)
            start_group(g0 + 2, 0, sem0)

            wait_group(g0 + 1, 1, sem1)
            tot = scan_group(1, tot)

            @pl.when(gg < PAIRS - 1)
            def _():
                start_group(g0 + 3, 1, sem1)

            return tot

        total = lax.fori_loop(0, PAIRS, pair_iter, total)
        wait_group(GROUPS - 1, 0, sem0)
        total = scan_group(0, total)

    out_v[:] = total
    pltpu.sync_copy(out_v, out_hbm.at[wid])


@functools.partial(
    pl.kernel,
    out_type=jax.ShapeDtypeStruct((NWORKERS, 16), jnp.int32),
    mesh=plsc.VectorSubcoreMesh(core_axis_name="c", subcore_axis_name="s"),
    scratch_types=[
        pltpu.VMEM((ACT,), jnp.float32),
        pltpu.VMEM((2 * ROWS, ACT), jnp.float32),
        pltpu.VMEM((16,), jnp.int32),
        pltpu.SemaphoreType.DMA,
        pltpu.SemaphoreType.DMA,
    ],
)
def _sc_argmax_sum(logits_hbm, noise_hbm, out_hbm, logits_v, noise_v, out_v,
                   sem0, sem1):
    _sc_body(logits_hbm, noise_hbm, out_hbm, logits_v, noise_v, out_v,
             sem0, sem1)


def _value_body(c0_ref, wlast_ref, ws_ref, Wv2_ref, bv2_ref, q_ref):
    total = jnp.sum(ws_ref[:].astype(jnp.float32)) / jnp.float32(NS)
    hv = jnp.tanh(c0_ref[:] + total * wlast_ref[:])
    q_ref[:] = (jnp.dot(hv, Wv2_ref[:], preferred_element_type=jnp.float32)
                + bv2_ref[:])


def _value(c0, wlast, ws, Wv2, bv2):
    return pl.pallas_call(
        _value_body,
        out_shape=jax.ShapeDtypeStruct((B, 1), jnp.float32),
    )(c0, wlast, ws, Wv2, bv2)


def kernel(obs, W1, b1, W2, b2, Wv1, bv1, Wv2, bv2, noise):
    logits, c0 = _policy(obs, W1, b1.reshape(1, HID), W2, b2.reshape(1, ACT),
                         Wv1[:OBS], bv1.reshape(1, HID))
    ws = _sc_argmax_sum(logits, noise)
    return _value(c0, Wv1[OBS:OBS + 1], ws, Wv2, bv2.reshape(1, 1))
